# baseline (speedup 1.0000x reference)
import functools

import jax
import jax.numpy as jnp
from jax import lax
from jax.experimental import pallas as pl
from jax.experimental.pallas import tpu as pltpu
from jax.experimental.pallas import tpu_sc as plsc

VOCAB = 1000000
B = 4096
SEQ = 200
D = 64
DP = 128
NC = 2
NS = 16
NW = NC * NS
BPW = B // NW
K = 100
CPW = BPW * 2
LANES = 16
CB = 1024


def _prep_body(tt_ref, out_ref):
    x = tt_ref[...] * jnp.float32(1.0 / SEQ)
    xt = jnp.transpose(x, (1, 0))
    out_ref[...] = jnp.concatenate([xt, xt], axis=1)


def _prep(tableT):
    grid = (pl.cdiv(VOCAB, CB),)
    return pl.pallas_call(
        _prep_body,
        grid=grid,
        in_specs=[pl.BlockSpec((D, CB), lambda i: (0, i))],
        out_specs=pl.BlockSpec((CB, DP), lambda i: (i, 0)),
        out_shape=jax.ShapeDtypeStruct((VOCAB, DP), jnp.float32),
    )(tableT)


def _pool_body(idx_hbm, table_hbm, out_hbm, idx_v, rows_v, acc_v, sem0, sem1):
    wid = lax.axis_index("s") * NC + lax.axis_index("c")
    pltpu.sync_copy(idx_hbm.at[wid], idx_v)

    sems = (sem0, sem1)
    pltpu.async_copy(table_hbm.at[idx_v.at[0]], rows_v.at[0], sems[0])
    pltpu.async_copy(table_hbm.at[idx_v.at[1]], rows_v.at[1], sems[1])

    def row_pair(i):
        acc = (jnp.zeros((LANES,), jnp.float32),) * 4

        def one_chunk(par, j, acc):
            pltpu.make_async_copy(
                table_hbm.at[idx_v.at[j]], rows_v.at[par], sems[par]
            ).wait()

            def rbody(r, a):
                a0, a1, a2, a3 = a
                a0 = a0 + rows_v[par, r, 0:16]
                a1 = a1 + rows_v[par, r, 16:32]
                a2 = a2 + rows_v[par, r, 32:48]
                a3 = a3 + rows_v[par, r, 48:64]
                return (a0, a1, a2, a3)

            acc = lax.fori_loop(0, K, rbody, acc, unroll=2)

            @pl.when(j + 2 < CPW)
            def _():
                pltpu.async_copy(
                    table_hbm.at[idx_v.at[j + 2]], rows_v.at[par], sems[par]
                )

            return acc

        acc = one_chunk(0, i * 2, acc)
        acc = one_chunk(1, i * 2 + 1, acc)
        acc_v[i, 0:16] = acc[0]
        acc_v[i, 16:32] = acc[1]
        acc_v[i, 32:48] = acc[2]
        acc_v[i, 48:64] = acc[3]

    pl.loop(0, BPW)(row_pair)

    pltpu.sync_copy(acc_v, out_hbm.at[pl.ds(wid * BPW, BPW)])


def _pool(idx, table_p):
    mesh = plsc.VectorSubcoreMesh(core_axis_name="c", subcore_axis_name="s")
    kfn = pl.kernel(
        _pool_body,
        mesh=mesh,
        compiler_params=pltpu.CompilerParams(use_tc_tiling_on_sc=False),
        out_type=jax.ShapeDtypeStruct((B, D), jnp.float32),
        scratch_types=[
            pltpu.VMEM((CPW, K), jnp.int32),
            pltpu.VMEM((2, K, DP), jnp.float32),
            pltpu.VMEM((BPW, D), jnp.float32),
            pltpu.SemaphoreType.DMA,
            pltpu.SemaphoreType.DMA,
        ],
    )
    return kfn(idx, table_p)


def _mlp_body(x_ref, w1_ref, b1_ref, w2_ref, b2_ref, wc_ref, bc_ref, o_ref):
    x = x_ref[...]
    h = jnp.dot(x, w1_ref[...], preferred_element_type=jnp.float32)
    h = jnp.maximum(h + b1_ref[...], 0.0)
    h = jnp.dot(h, w2_ref[...], preferred_element_type=jnp.float32)
    h = jnp.maximum(h + b2_ref[...], 0.0)
    o = jnp.dot(h, wc_ref[...], preferred_element_type=jnp.float32)
    o_ref[...] = jax.nn.sigmoid(o + bc_ref[...])


def _mlp(x, W1, b1, W2, b2, Wc, bc):
    bm = 512
    grid = (B // bm,)
    return pl.pallas_call(
        _mlp_body,
        grid=grid,
        in_specs=[
            pl.BlockSpec((bm, D), lambda i: (i, 0)),
            pl.BlockSpec(W1.shape, lambda i: (0, 0)),
            pl.BlockSpec(b1.shape, lambda i: (0, 0)),
            pl.BlockSpec(W2.shape, lambda i: (0, 0)),
            pl.BlockSpec(b2.shape, lambda i: (0, 0)),
            pl.BlockSpec(Wc.shape, lambda i: (0, 0)),
            pl.BlockSpec(bc.shape, lambda i: (0, 0)),
        ],
        out_specs=pl.BlockSpec((bm, Wc.shape[1]), lambda i: (i, 0)),
        out_shape=jax.ShapeDtypeStruct((B, Wc.shape[1]), jnp.float32),
    )(x, W1, b1, W2, b2, Wc, bc)


@jax.jit
def kernel(inputs, table, W1, b1, W2, b2, Wc, bc):
    idx = inputs.astype(jnp.int32).reshape(NW, CPW, K)
    table_p = _prep(table.T)
    pooled = _pool(idx, table_p)
    out = _mlp(
        pooled,
        W1,
        b1.reshape(1, -1),
        W2,
        b2.reshape(1, -1),
        Wc,
        bc.reshape(1, -1),
    )
    return out


# speedup vs baseline: 1.5674x; 1.5674x over previous
import functools

import jax
import jax.numpy as jnp
from jax import lax
from jax.experimental import pallas as pl
from jax.experimental.pallas import tpu as pltpu
from jax.experimental.pallas import tpu_sc as plsc

VOCAB = 1000000
B = 4096
SEQ = 200
D = 64
DP = 128
NC = 2
NS = 16
NW = NC * NS
BPW = B // NW
K = 100
CPW = BPW * 2
LANES = 16
CB = 4096


def _prep_body(tt_ref, out_ref):
    x = tt_ref[...]
    i0 = lax.broadcasted_iota(jnp.int32, (D, D), 0)
    i1 = lax.broadcasted_iota(jnp.int32, (D, D), 1)
    ident = jnp.where(i0 == i1, jnp.float32(1.0 / SEQ), 0.0)
    xt = lax.dot_general(
        x, ident, (((0,), (0,)), ((), ())),
        preferred_element_type=jnp.float32,
    )
    out_ref[...] = jnp.concatenate([xt, xt], axis=1)


def _prep(tableT):
    grid = (pl.cdiv(VOCAB, CB),)
    return pl.pallas_call(
        _prep_body,
        grid=grid,
        in_specs=[pl.BlockSpec((D, CB), lambda i: (0, i))],
        out_specs=pl.BlockSpec((CB, DP), lambda i: (i, 0)),
        out_shape=jax.ShapeDtypeStruct((VOCAB, DP), jnp.float32),
    )(tableT)


def _pool_body(idx_hbm, table_hbm, out_hbm, idx_v, rows_v, acc_v, sem0, sem1):
    wid = lax.axis_index("s") * NC + lax.axis_index("c")
    pltpu.sync_copy(idx_hbm.at[wid], idx_v)

    sems = (sem0, sem1)
    pltpu.async_copy(table_hbm.at[idx_v.at[0]], rows_v.at[0], sems[0])
    pltpu.async_copy(table_hbm.at[idx_v.at[1]], rows_v.at[1], sems[1])

    def row_pair(i):
        acc = (jnp.zeros((LANES,), jnp.float32),) * 4

        def one_chunk(par, j, acc):
            pltpu.make_async_copy(
                table_hbm.at[idx_v.at[j]], rows_v.at[par], sems[par]
            ).wait()

            def rbody(r, a):
                a0, a1, a2, a3 = a
                a0 = a0 + rows_v[par, r, 0:16]
                a1 = a1 + rows_v[par, r, 16:32]
                a2 = a2 + rows_v[par, r, 32:48]
                a3 = a3 + rows_v[par, r, 48:64]
                return (a0, a1, a2, a3)

            acc = lax.fori_loop(0, K, rbody, acc, unroll=2)

            @pl.when(j + 2 < CPW)
            def _():
                pltpu.async_copy(
                    table_hbm.at[idx_v.at[j + 2]], rows_v.at[par], sems[par]
                )

            return acc

        acc = one_chunk(0, i * 2, acc)
        acc = one_chunk(1, i * 2 + 1, acc)
        acc_v[i, 0:16] = acc[0]
        acc_v[i, 16:32] = acc[1]
        acc_v[i, 32:48] = acc[2]
        acc_v[i, 48:64] = acc[3]

    pl.loop(0, BPW)(row_pair)

    pltpu.sync_copy(acc_v, out_hbm.at[pl.ds(wid * BPW, BPW)])


def _pool(idx, table_p):
    mesh = plsc.VectorSubcoreMesh(core_axis_name="c", subcore_axis_name="s")
    kfn = pl.kernel(
        _pool_body,
        mesh=mesh,
        compiler_params=pltpu.CompilerParams(use_tc_tiling_on_sc=False),
        out_type=jax.ShapeDtypeStruct((B, D), jnp.float32),
        scratch_types=[
            pltpu.VMEM((CPW, K), jnp.int32),
            pltpu.VMEM((2, K, DP), jnp.float32),
            pltpu.VMEM((BPW, D), jnp.float32),
            pltpu.SemaphoreType.DMA,
            pltpu.SemaphoreType.DMA,
        ],
    )
    return kfn(idx, table_p)


def _mlp_body(x_ref, w1_ref, b1_ref, w2_ref, b2_ref, wc_ref, bc_ref, o_ref):
    x = x_ref[...]
    h = jnp.dot(x, w1_ref[...], preferred_element_type=jnp.float32)
    h = jnp.maximum(h + b1_ref[...], 0.0)
    h = jnp.dot(h, w2_ref[...], preferred_element_type=jnp.float32)
    h = jnp.maximum(h + b2_ref[...], 0.0)
    o = jnp.dot(h, wc_ref[...], preferred_element_type=jnp.float32)
    o_ref[...] = jax.nn.sigmoid(o + bc_ref[...])


def _mlp(x, W1, b1, W2, b2, Wc, bc):
    bm = 512
    grid = (B // bm,)
    return pl.pallas_call(
        _mlp_body,
        grid=grid,
        in_specs=[
            pl.BlockSpec((bm, D), lambda i: (i, 0)),
            pl.BlockSpec(W1.shape, lambda i: (0, 0)),
            pl.BlockSpec(b1.shape, lambda i: (0, 0)),
            pl.BlockSpec(W2.shape, lambda i: (0, 0)),
            pl.BlockSpec(b2.shape, lambda i: (0, 0)),
            pl.BlockSpec(Wc.shape, lambda i: (0, 0)),
            pl.BlockSpec(bc.shape, lambda i: (0, 0)),
        ],
        out_specs=pl.BlockSpec((bm, Wc.shape[1]), lambda i: (i, 0)),
        out_shape=jax.ShapeDtypeStruct((B, Wc.shape[1]), jnp.float32),
    )(x, W1, b1, W2, b2, Wc, bc)


@jax.jit
def kernel(inputs, table, W1, b1, W2, b2, Wc, bc):
    idx = inputs.astype(jnp.int32).reshape(NW, CPW, K)
    table_p = _prep(table.T)
    pooled = _pool(idx, table_p)
    out = _mlp(
        pooled,
        W1,
        b1.reshape(1, -1),
        W2,
        b2.reshape(1, -1),
        Wc,
        bc.reshape(1, -1),
    )
    return out


# speedup vs baseline: 1.8589x; 1.1859x over previous
import functools

import jax
import jax.numpy as jnp
from jax import lax
from jax.experimental import pallas as pl
from jax.experimental.pallas import tpu as pltpu
from jax.experimental.pallas import tpu_sc as plsc

VOCAB = 1000000
B = 4096
SEQ = 200
D = 64
DP = 128
NC = 2
NS = 16
NW = NC * NS
BPW = B // NW
K = 100
CPW = BPW * 2
LANES = 16
CB = 16384


def _prep_body(tt_ref, out_ref):
    x = tt_ref[...]
    i0 = lax.broadcasted_iota(jnp.int32, (D, D), 0)
    i1 = lax.broadcasted_iota(jnp.int32, (D, D), 1)
    ident = jnp.where(i0 == i1, jnp.float32(1.0 / SEQ), 0.0)
    xt = lax.dot_general(
        x, ident, (((0,), (0,)), ((), ())),
        preferred_element_type=jnp.float32,
    )
    out_ref[...] = jnp.concatenate([xt, xt], axis=1)


def _prep(tableT):
    grid = (pl.cdiv(VOCAB, CB),)
    return pl.pallas_call(
        _prep_body,
        grid=grid,
        in_specs=[pl.BlockSpec((D, CB), lambda i: (0, i))],
        out_specs=pl.BlockSpec((CB, DP), lambda i: (i, 0)),
        out_shape=jax.ShapeDtypeStruct((VOCAB, DP), jnp.float32),
    )(tableT)


def _pool_body(idx_hbm, table_hbm, out_hbm, idx_v, rows_v, acc_v, sem0, sem1):
    wid = lax.axis_index("s") * NC + lax.axis_index("c")
    pltpu.sync_copy(idx_hbm.at[wid], idx_v)

    sems = (sem0, sem1)
    pltpu.async_copy(table_hbm.at[idx_v.at[0]], rows_v.at[0], sems[0])
    pltpu.async_copy(table_hbm.at[idx_v.at[1]], rows_v.at[1], sems[1])

    def row_pair(i):
        acc = (jnp.zeros((LANES,), jnp.float32),) * 4

        def one_chunk(par, j, acc):
            pltpu.make_async_copy(
                table_hbm.at[idx_v.at[j]], rows_v.at[par], sems[par]
            ).wait()

            def rbody(r, a):
                a0, a1, a2, a3 = a
                a0 = a0 + rows_v[par, r, 0:16]
                a1 = a1 + rows_v[par, r, 16:32]
                a2 = a2 + rows_v[par, r, 32:48]
                a3 = a3 + rows_v[par, r, 48:64]
                return (a0, a1, a2, a3)

            acc = lax.fori_loop(0, K, rbody, acc, unroll=2)

            @pl.when(j + 2 < CPW)
            def _():
                pltpu.async_copy(
                    table_hbm.at[idx_v.at[j + 2]], rows_v.at[par], sems[par]
                )

            return acc

        acc = one_chunk(0, i * 2, acc)
        acc = one_chunk(1, i * 2 + 1, acc)
        acc_v[i, 0:16] = acc[0]
        acc_v[i, 16:32] = acc[1]
        acc_v[i, 32:48] = acc[2]
        acc_v[i, 48:64] = acc[3]

    pl.loop(0, BPW)(row_pair)

    pltpu.sync_copy(acc_v, out_hbm.at[pl.ds(wid * BPW, BPW)])


def _pool(idx, table_p):
    mesh = plsc.VectorSubcoreMesh(core_axis_name="c", subcore_axis_name="s")
    kfn = pl.kernel(
        _pool_body,
        mesh=mesh,
        compiler_params=pltpu.CompilerParams(use_tc_tiling_on_sc=False),
        out_type=jax.ShapeDtypeStruct((B, D), jnp.float32),
        scratch_types=[
            pltpu.VMEM((CPW, K), jnp.int32),
            pltpu.VMEM((2, K, DP), jnp.float32),
            pltpu.VMEM((BPW, D), jnp.float32),
            pltpu.SemaphoreType.DMA,
            pltpu.SemaphoreType.DMA,
        ],
    )
    return kfn(idx, table_p)


def _mlp_body(x_ref, w1_ref, b1_ref, w2_ref, b2_ref, wc_ref, bc_ref, o_ref):
    x = x_ref[...]
    h = jnp.dot(x, w1_ref[...], preferred_element_type=jnp.float32)
    h = jnp.maximum(h + b1_ref[...], 0.0)
    h = jnp.dot(h, w2_ref[...], preferred_element_type=jnp.float32)
    h = jnp.maximum(h + b2_ref[...], 0.0)
    o = jnp.dot(h, wc_ref[...], preferred_element_type=jnp.float32)
    o_ref[...] = jax.nn.sigmoid(o + bc_ref[...])


def _mlp(x, W1, b1, W2, b2, Wc, bc):
    bm = 512
    grid = (B // bm,)
    return pl.pallas_call(
        _mlp_body,
        grid=grid,
        in_specs=[
            pl.BlockSpec((bm, D), lambda i: (i, 0)),
            pl.BlockSpec(W1.shape, lambda i: (0, 0)),
            pl.BlockSpec(b1.shape, lambda i: (0, 0)),
            pl.BlockSpec(W2.shape, lambda i: (0, 0)),
            pl.BlockSpec(b2.shape, lambda i: (0, 0)),
            pl.BlockSpec(Wc.shape, lambda i: (0, 0)),
            pl.BlockSpec(bc.shape, lambda i: (0, 0)),
        ],
        out_specs=pl.BlockSpec((bm, Wc.shape[1]), lambda i: (i, 0)),
        out_shape=jax.ShapeDtypeStruct((B, Wc.shape[1]), jnp.float32),
    )(x, W1, b1, W2, b2, Wc, bc)


@jax.jit
def kernel(inputs, table, W1, b1, W2, b2, Wc, bc):
    idx = inputs.astype(jnp.int32).reshape(NW, CPW, K)
    table_p = _prep(table.T)
    pooled = _pool(idx, table_p)
    out = _mlp(
        pooled,
        W1,
        b1.reshape(1, -1),
        W2,
        b2.reshape(1, -1),
        Wc,
        bc.reshape(1, -1),
    )
    return out


# speedup vs baseline: 1.8978x; 1.0209x over previous
import functools

import jax
import jax.numpy as jnp
from jax import lax
from jax.experimental import pallas as pl
from jax.experimental.pallas import tpu as pltpu
from jax.experimental.pallas import tpu_sc as plsc

VOCAB = 1000000
B = 4096
SEQ = 200
D = 64
DP = 128
NC = 2
NS = 16
NW = NC * NS
BPW = B // NW
K = 100
CPW = BPW * 2
LANES = 16
CB = 24576


def _prep_body(tt_ref, out_ref):
    x = tt_ref[...]
    i0 = lax.broadcasted_iota(jnp.int32, (D, D), 0)
    i1 = lax.broadcasted_iota(jnp.int32, (D, D), 1)
    ident = jnp.where(i0 == i1, jnp.float32(1.0 / SEQ), 0.0)
    xt = lax.dot_general(
        x, ident, (((0,), (0,)), ((), ())),
        preferred_element_type=jnp.float32,
    )
    out_ref[...] = jnp.concatenate([xt, xt], axis=1)


def _prep(tableT):
    grid = (pl.cdiv(VOCAB, CB),)
    return pl.pallas_call(
        _prep_body,
        grid=grid,
        in_specs=[pl.BlockSpec((D, CB), lambda i: (0, i))],
        out_specs=pl.BlockSpec((CB, DP), lambda i: (i, 0)),
        out_shape=jax.ShapeDtypeStruct((VOCAB, DP), jnp.float32),
    )(tableT)


def _pool_body(idx_hbm, table_hbm, out_hbm, idx_v, rows_v, acc_v, sem0, sem1):
    wid = lax.axis_index("s") * NC + lax.axis_index("c")
    pltpu.sync_copy(idx_hbm.at[wid], idx_v)

    sems = (sem0, sem1)
    pltpu.async_copy(table_hbm.at[idx_v.at[0]], rows_v.at[0], sems[0])
    pltpu.async_copy(table_hbm.at[idx_v.at[1]], rows_v.at[1], sems[1])

    def row_pair(i):
        acc = (jnp.zeros((LANES,), jnp.float32),) * 4

        def one_chunk(par, j, acc):
            pltpu.make_async_copy(
                table_hbm.at[idx_v.at[j]], rows_v.at[par], sems[par]
            ).wait()

            def rbody(r, a):
                a0, a1, a2, a3 = a
                a0 = a0 + rows_v[par, r, 0:16]
                a1 = a1 + rows_v[par, r, 16:32]
                a2 = a2 + rows_v[par, r, 32:48]
                a3 = a3 + rows_v[par, r, 48:64]
                return (a0, a1, a2, a3)

            acc = lax.fori_loop(0, K, rbody, acc, unroll=2)

            @pl.when(j + 2 < CPW)
            def _():
                pltpu.async_copy(
                    table_hbm.at[idx_v.at[j + 2]], rows_v.at[par], sems[par]
                )

            return acc

        acc = one_chunk(0, i * 2, acc)
        acc = one_chunk(1, i * 2 + 1, acc)
        acc_v[i, 0:16] = acc[0]
        acc_v[i, 16:32] = acc[1]
        acc_v[i, 32:48] = acc[2]
        acc_v[i, 48:64] = acc[3]

    pl.loop(0, BPW)(row_pair)

    pltpu.sync_copy(acc_v, out_hbm.at[pl.ds(wid * BPW, BPW)])


def _pool(idx, table_p):
    mesh = plsc.VectorSubcoreMesh(core_axis_name="c", subcore_axis_name="s")
    kfn = pl.kernel(
        _pool_body,
        mesh=mesh,
        compiler_params=pltpu.CompilerParams(use_tc_tiling_on_sc=False),
        out_type=jax.ShapeDtypeStruct((B, D), jnp.float32),
        scratch_types=[
            pltpu.VMEM((CPW, K), jnp.int32),
            pltpu.VMEM((2, K, DP), jnp.float32),
            pltpu.VMEM((BPW, D), jnp.float32),
            pltpu.SemaphoreType.DMA,
            pltpu.SemaphoreType.DMA,
        ],
    )
    return kfn(idx, table_p)


def _mlp_body(x_ref, w1_ref, b1_ref, w2_ref, b2_ref, wc_ref, bc_ref, o_ref):
    x = x_ref[...]
    h = jnp.dot(x, w1_ref[...], preferred_element_type=jnp.float32)
    h = jnp.maximum(h + b1_ref[...], 0.0)
    h = jnp.dot(h, w2_ref[...], preferred_element_type=jnp.float32)
    h = jnp.maximum(h + b2_ref[...], 0.0)
    o = jnp.dot(h, wc_ref[...], preferred_element_type=jnp.float32)
    o_ref[...] = jax.nn.sigmoid(o + bc_ref[...])


def _mlp(x, W1, b1, W2, b2, Wc, bc):
    bm = 512
    grid = (B // bm,)
    return pl.pallas_call(
        _mlp_body,
        grid=grid,
        in_specs=[
            pl.BlockSpec((bm, D), lambda i: (i, 0)),
            pl.BlockSpec(W1.shape, lambda i: (0, 0)),
            pl.BlockSpec(b1.shape, lambda i: (0, 0)),
            pl.BlockSpec(W2.shape, lambda i: (0, 0)),
            pl.BlockSpec(b2.shape, lambda i: (0, 0)),
            pl.BlockSpec(Wc.shape, lambda i: (0, 0)),
            pl.BlockSpec(bc.shape, lambda i: (0, 0)),
        ],
        out_specs=pl.BlockSpec((bm, Wc.shape[1]), lambda i: (i, 0)),
        out_shape=jax.ShapeDtypeStruct((B, Wc.shape[1]), jnp.float32),
    )(x, W1, b1, W2, b2, Wc, bc)


@jax.jit
def kernel(inputs, table, W1, b1, W2, b2, Wc, bc):
    idx = inputs.astype(jnp.int32).reshape(NW, CPW, K)
    table_p = _prep(table.T)
    pooled = _pool(idx, table_p)
    out = _mlp(
        pooled,
        W1,
        b1.reshape(1, -1),
        W2,
        b2.reshape(1, -1),
        Wc,
        bc.reshape(1, -1),
    )
    return out


# speedup vs baseline: 2.4368x; 1.2841x over previous
import functools

import jax
import jax.numpy as jnp
from jax import lax
from jax.experimental import pallas as pl
from jax.experimental.pallas import tpu as pltpu
from jax.experimental.pallas import tpu_sc as plsc

VOCAB = 1000000
B = 4096
SEQ = 200
D = 64
DP = 128
NC = 2
NS = 16
NW = NC * NS
BPW = B // NW
K = 100
CPW = BPW * 2
LANES = 16
CBH = 8064
HALF = 499968


def _prep_body(lt_ref, rt_ref, out_ref):
    i0 = lax.broadcasted_iota(jnp.int32, (D, D), 0)
    i1 = lax.broadcasted_iota(jnp.int32, (D, D), 1)
    ident = jnp.where(i0 == i1, jnp.float32(1.0 / SEQ), 0.0)
    dims = (((0,), (0,)), ((), ()))
    xl = lax.dot_general(lt_ref[...], ident, dims,
                         preferred_element_type=jnp.float32)
    xr = lax.dot_general(rt_ref[...], ident, dims,
                         preferred_element_type=jnp.float32)
    out_ref[...] = jnp.concatenate([xl, xr], axis=1)


def _prep(tableT):
    nblk = HALF // CBH
    grid = (nblk + 1,)

    def lmap(i):
        return (0, jnp.where(i < nblk, i, 2 * nblk))

    def rmap(i):
        return (0, jnp.where(i < nblk, i + nblk, 2 * nblk))

    return pl.pallas_call(
        _prep_body,
        grid=grid,
        in_specs=[
            pl.BlockSpec((D, CBH), lmap),
            pl.BlockSpec((D, CBH), rmap),
        ],
        out_specs=pl.BlockSpec((CBH, DP), lambda i: (i, 0)),
        out_shape=jax.ShapeDtypeStruct(((nblk + 1) * CBH, DP), jnp.float32),
    )(tableT, tableT)


def _pool_body(idx_hbm, table_hbm, out_hbm, idx_v, rows_v, acc_v, sem0, sem1):
    wid = lax.axis_index("s") * NC + lax.axis_index("c")
    pltpu.sync_copy(idx_hbm.at[wid], idx_v)

    sems = (sem0, sem1)
    pltpu.async_copy(table_hbm.at[idx_v.at[0]], rows_v.at[0], sems[0])
    pltpu.async_copy(table_hbm.at[idx_v.at[1]], rows_v.at[1], sems[1])

    def row_pair(i):
        acc = (jnp.zeros((LANES,), jnp.float32),) * 4

        def one_chunk(par, j, acc):
            pltpu.make_async_copy(
                table_hbm.at[idx_v.at[j]], rows_v.at[par], sems[par]
            ).wait()

            def rbody(r, a):
                a0, a1, a2, a3 = a
                a0 = a0 + rows_v[par, r, 0:16]
                a1 = a1 + rows_v[par, r, 16:32]
                a2 = a2 + rows_v[par, r, 32:48]
                a3 = a3 + rows_v[par, r, 48:64]
                return (a0, a1, a2, a3)

            acc = lax.fori_loop(0, K, rbody, acc, unroll=2)

            @pl.when(j + 2 < CPW)
            def _():
                pltpu.async_copy(
                    table_hbm.at[idx_v.at[j + 2]], rows_v.at[par], sems[par]
                )

            return acc

        acc = one_chunk(0, i * 2, acc)
        acc = one_chunk(1, i * 2 + 1, acc)
        acc_v[i, 0:16] = acc[0]
        acc_v[i, 16:32] = acc[1]
        acc_v[i, 32:48] = acc[2]
        acc_v[i, 48:64] = acc[3]

    pl.loop(0, BPW)(row_pair)

    pltpu.sync_copy(acc_v, out_hbm.at[pl.ds(wid * BPW, BPW)])


def _pool(idx, table_p):
    mesh = plsc.VectorSubcoreMesh(core_axis_name="c", subcore_axis_name="s")
    kfn = pl.kernel(
        _pool_body,
        mesh=mesh,
        compiler_params=pltpu.CompilerParams(use_tc_tiling_on_sc=False),
        out_type=jax.ShapeDtypeStruct((B, D), jnp.float32),
        scratch_types=[
            pltpu.VMEM((CPW, K), jnp.int32),
            pltpu.VMEM((2, K, D), jnp.float32),
            pltpu.VMEM((BPW, D), jnp.float32),
            pltpu.SemaphoreType.DMA,
            pltpu.SemaphoreType.DMA,
        ],
    )
    return kfn(idx, table_p)


def _mlp_body(x_ref, w1_ref, b1_ref, w2_ref, b2_ref, wc_ref, bc_ref, o_ref):
    x = x_ref[...]
    h = jnp.dot(x, w1_ref[...], preferred_element_type=jnp.float32)
    h = jnp.maximum(h + b1_ref[...], 0.0)
    h = jnp.dot(h, w2_ref[...], preferred_element_type=jnp.float32)
    h = jnp.maximum(h + b2_ref[...], 0.0)
    o = jnp.dot(h, wc_ref[...], preferred_element_type=jnp.float32)
    o_ref[...] = jax.nn.sigmoid(o + bc_ref[...])


def _mlp(x, W1, b1, W2, b2, Wc, bc):
    bm = 512
    grid = (B // bm,)
    return pl.pallas_call(
        _mlp_body,
        grid=grid,
        in_specs=[
            pl.BlockSpec((bm, D), lambda i: (i, 0)),
            pl.BlockSpec(W1.shape, lambda i: (0, 0)),
            pl.BlockSpec(b1.shape, lambda i: (0, 0)),
            pl.BlockSpec(W2.shape, lambda i: (0, 0)),
            pl.BlockSpec(b2.shape, lambda i: (0, 0)),
            pl.BlockSpec(Wc.shape, lambda i: (0, 0)),
            pl.BlockSpec(bc.shape, lambda i: (0, 0)),
        ],
        out_specs=pl.BlockSpec((bm, Wc.shape[1]), lambda i: (i, 0)),
        out_shape=jax.ShapeDtypeStruct((B, Wc.shape[1]), jnp.float32),
    )(x, W1, b1, W2, b2, Wc, bc)


@jax.jit
def kernel(inputs, table, W1, b1, W2, b2, Wc, bc):
    i = inputs.astype(jnp.int32)
    q = jnp.where(
        i < HALF,
        2 * i,
        jnp.where(i < 2 * HALF, 2 * i - (2 * HALF - 1), 2 * i - 2 * HALF),
    )
    idx = q.reshape(NW, CPW, K)
    nrows = 2 * (HALF // CBH + 1) * CBH
    table_p = _prep(table.T).reshape(nrows, D)
    pooled = _pool(idx, table_p)
    out = _mlp(
        pooled,
        W1,
        b1.reshape(1, -1),
        W2,
        b2.reshape(1, -1),
        Wc,
        bc.reshape(1, -1),
    )
    return out
